# baseline (speedup 1.0000x reference)
import functools

import jax
import jax.numpy as jnp
import numpy as np
from jax import lax
from jax.experimental import pallas as pl
from jax.experimental.pallas import tpu as pltpu
from jax.experimental.pallas import tpu_sc as plsc

N_BIN = 16
OUT_BITS = 22
MASK22 = (1 << OUT_BITS) - 1
BATCH_DIM = 16384
HASH_K32 = np.int32(np.uint32(2654435761))

NC = 2
NS = 16
NW = NC * NS
CHUNK = 1024
KEY_RANGE = 200000

TC_BR = 4096


def _loop_i32(n, body):
    def wbody(c, _):
        body(c)
        return c + np.int32(1), None
    lax.scan(wbody, np.int32(0), None, length=n)


def _sc_gather(keys, table):
    n = keys.shape[0]
    npw = n // NW
    n_chunks = npw // CHUNK
    assert n == NW * npw and npw == n_chunks * CHUNK and n_chunks % 2 == 0

    mesh = plsc.VectorSubcoreMesh(core_axis_name="c", subcore_axis_name="s")

    @functools.partial(
        pl.kernel,
        out_type=jax.ShapeDtypeStruct((n, N_BIN), jnp.float32),
        mesh=mesh,
        compiler_params=pltpu.CompilerParams(use_tc_tiling_on_sc=False),
        scratch_types=[
            pltpu.VMEM((CHUNK,), jnp.int32),
            pltpu.VMEM((CHUNK,), jnp.int32),
            pltpu.VMEM((CHUNK, N_BIN), jnp.float32),
            pltpu.VMEM((CHUNK, N_BIN), jnp.float32),
            pltpu.SemaphoreType.DMA,
            pltpu.SemaphoreType.DMA,
            pltpu.SemaphoreType.DMA,
        ],
    )
    def k(keys_hbm, table_hbm, rows_hbm,
          keys_a, keys_b, rows_a, rows_b, sem_k, sem_g, sem_w):
        wid = (lax.axis_index("c").astype(jnp.int32) * np.int32(NS)
               + lax.axis_index("s").astype(jnp.int32))
        wbase = wid * np.int32(npw)

        ck0 = pltpu.async_copy(keys_hbm.at[pl.ds(wbase, CHUNK)], keys_a, sem_k)

        def pair_body(c):
            a = c * np.int32(2)
            base_a = wbase + a * np.int32(CHUNK)
            base_b = base_a + np.int32(CHUNK)
            nxt = jnp.where(c == np.int32(n_chunks // 2 - 1),
                            np.int32(0), a + np.int32(2))
            base_n = wbase + nxt * np.int32(CHUNK)

            ck0.wait()
            ga = pltpu.async_copy(table_hbm.at[keys_a], rows_a, sem_g)
            ckb = pltpu.async_copy(keys_hbm.at[pl.ds(base_b, CHUNK)],
                                   keys_b, sem_k)
            ckb.wait()
            gb = pltpu.async_copy(table_hbm.at[keys_b], rows_b, sem_g)
            ga.wait()
            wa = pltpu.async_copy(rows_a, rows_hbm.at[pl.ds(base_a, CHUNK)],
                                  sem_w)
            pltpu.async_copy(keys_hbm.at[pl.ds(base_n, CHUNK)], keys_a, sem_k)
            gb.wait()
            wb = pltpu.async_copy(rows_b, rows_hbm.at[pl.ds(base_b, CHUNK)],
                                  sem_w)
            wa.wait()
            wb.wait()

        _loop_i32(n_chunks // 2, pair_body)
        pltpu.make_async_copy(keys_hbm.at[pl.ds(wbase, CHUNK)],
                              keys_a, sem_k).wait()

    return k(keys, table)


def _tc_bin_hash(rows128, keys8, vals8, n_feat):
    m = rows128.shape[0]
    grid = m // TC_BR
    assert grid * TC_BR == m

    def body(rows_ref, keys_ref, vals_ref, outk_ref, outv_ref):
        rows = rows_ref[...]
        vals = vals_ref[...]
        keys = keys_ref[...]
        e_exp = (lax.broadcasted_iota(jnp.int32, (8, 128), 1) // np.int32(16)
                 == lax.broadcasted_iota(jnp.int32, (8, 128), 0)
                 ).astype(jnp.float32)
        g_sum = (lax.broadcasted_iota(jnp.int32, (128, 8), 0) // np.int32(16)
                 == lax.broadcasted_iota(jnp.int32, (128, 8), 1)
                 ).astype(jnp.float32)
        vals_exp = lax.dot_general(
            vals, e_exp, (((1,), (0,)), ((), ())),
            precision=lax.Precision.HIGHEST)
        cmpf = jnp.where(rows < vals_exp, jnp.float32(1.0), jnp.float32(0.0))
        cnt = lax.dot_general(
            cmpf, g_sum, (((1,), (0,)), ((), ())),
            precision=lax.Precision.HIGHEST).astype(jnp.int32)
        h = (keys * HASH_K32 + cnt) * HASH_K32
        found = keys < np.int32(n_feat)
        outk_ref[...] = jnp.where(found, h & np.int32(MASK22),
                                  keys & np.int32(MASK22))
        outv_ref[...] = jnp.where(found, jnp.float32(1.0), vals)

    return pl.pallas_call(
        body,
        grid=(grid,),
        in_specs=[
            pl.BlockSpec((TC_BR, 128), lambda i: (i, np.int32(0))),
            pl.BlockSpec((TC_BR, 8), lambda i: (i, np.int32(0))),
            pl.BlockSpec((TC_BR, 8), lambda i: (i, np.int32(0))),
        ],
        out_specs=[
            pl.BlockSpec((TC_BR, 8), lambda i: (i, np.int32(0))),
            pl.BlockSpec((TC_BR, 8), lambda i: (i, np.int32(0))),
        ],
        out_shape=[
            jax.ShapeDtypeStruct((m, 8), jnp.int32),
            jax.ShapeDtypeStruct((m, 8), jnp.float32),
        ],
    )(rows128, keys8, vals8)


def kernel(input_ids, input_keys, input_vals, feature_ids, bin_vals):
    n = input_keys.shape[0]
    n_feat = feature_ids.shape[0]
    keys32 = input_keys.astype(jnp.int32)
    keys_idx = jnp.minimum(keys32, np.int32(n_feat - 1))
    table = bin_vals.reshape(n_feat, N_BIN)
    rows = _sc_gather(keys_idx, table)
    outk8, outv8 = _tc_bin_hash(
        rows.reshape(n // 8, 8 * N_BIN),
        keys32.reshape(n // 8, 8),
        input_vals.reshape(n // 8, 8),
        n_feat,
    )
    out_keys = outk8.reshape(n).astype(jnp.int64)
    out_vals = outv8.reshape(n)
    dense_shape = jnp.array([BATCH_DIM, 1 << OUT_BITS], dtype=jnp.int64)
    return out_keys, out_vals, input_ids, dense_shape


# speedup vs baseline: 3.6138x; 3.6138x over previous
import functools

import jax
import jax.numpy as jnp
import numpy as np
from jax import lax
from jax.experimental import pallas as pl
from jax.experimental.pallas import tpu as pltpu
from jax.experimental.pallas import tpu_sc as plsc

N_BIN = 16
OUT_BITS = 22
MASK22 = (1 << OUT_BITS) - 1
BATCH_DIM = 16384
HASH_K32 = np.int32(np.uint32(2654435761))

NC = 2
NS = 16
NW = NC * NS
CHUNK = 1024
KEY_RANGE = 200000

TC_BR = 4096


def _loop_i32(n, body):
    def wbody(c, _):
        body(c)
        return c + np.int32(1), None
    lax.scan(wbody, np.int32(0), None, length=n)


def _sc_gather(keys, table_padded):
    n = keys.shape[0]
    npw = n // NW
    n_chunks = npw // CHUNK
    assert n == NW * npw and npw == n_chunks * CHUNK and n_chunks % 2 == 0

    mesh = plsc.VectorSubcoreMesh(core_axis_name="c", subcore_axis_name="s")

    @functools.partial(
        pl.kernel,
        out_type=jax.ShapeDtypeStruct((n, N_BIN), jnp.float32),
        mesh=mesh,
        compiler_params=pltpu.CompilerParams(use_tc_tiling_on_sc=False),
        scratch_types=[
            pltpu.VMEM((CHUNK,), jnp.int32),
            pltpu.VMEM((CHUNK,), jnp.int32),
            pltpu.VMEM((CHUNK, N_BIN), jnp.float32),
            pltpu.VMEM((CHUNK, N_BIN), jnp.float32),
            pltpu.SemaphoreType.DMA,
            pltpu.SemaphoreType.DMA,
            pltpu.SemaphoreType.DMA,
        ],
    )
    def k(keys_hbm, table_hbm, rows_hbm,
          keys_a, keys_b, rows_a, rows_b, sem_k, sem_g, sem_w):
        wid = (lax.axis_index("c").astype(jnp.int32) * np.int32(NS)
               + lax.axis_index("s").astype(jnp.int32))
        wbase = wid * np.int32(npw)

        ck0 = pltpu.async_copy(keys_hbm.at[pl.ds(wbase, CHUNK)], keys_a, sem_k)

        def pair_body(c):
            a = c * np.int32(2)
            base_a = wbase + a * np.int32(CHUNK)
            base_b = base_a + np.int32(CHUNK)
            nxt = jnp.where(c == np.int32(n_chunks // 2 - 1),
                            np.int32(0), a + np.int32(2))
            base_n = wbase + nxt * np.int32(CHUNK)

            ck0.wait()
            ga = pltpu.async_copy(table_hbm.at[keys_a], rows_a, sem_g)
            ckb = pltpu.async_copy(keys_hbm.at[pl.ds(base_b, CHUNK)],
                                   keys_b, sem_k)
            ckb.wait()
            gb = pltpu.async_copy(table_hbm.at[keys_b], rows_b, sem_g)
            ga.wait()
            wa = pltpu.async_copy(rows_a, rows_hbm.at[pl.ds(base_a, CHUNK)],
                                  sem_w)
            pltpu.async_copy(keys_hbm.at[pl.ds(base_n, CHUNK)], keys_a, sem_k)
            gb.wait()
            wb = pltpu.async_copy(rows_b, rows_hbm.at[pl.ds(base_b, CHUNK)],
                                  sem_w)
            wa.wait()
            wb.wait()

        _loop_i32(n_chunks // 2, pair_body)
        pltpu.make_async_copy(keys_hbm.at[pl.ds(wbase, CHUNK)],
                              keys_a, sem_k).wait()

    return k(keys, table_padded)


def _tc_bin_hash(rows128, keys8, vals8, n_feat):
    m = rows128.shape[0]
    grid = m // TC_BR
    assert grid * TC_BR == m

    def body(rows_ref, keys_ref, vals_ref, outk_ref, outv_ref):
        rows = rows_ref[...]
        vals = vals_ref[...]
        keys = keys_ref[...]
        e_exp = (lax.broadcasted_iota(jnp.int32, (8, 128), 1) // np.int32(16)
                 == lax.broadcasted_iota(jnp.int32, (8, 128), 0)
                 ).astype(jnp.float32)
        g_sum = (lax.broadcasted_iota(jnp.int32, (128, 8), 0) // np.int32(16)
                 == lax.broadcasted_iota(jnp.int32, (128, 8), 1)
                 ).astype(jnp.float32)
        vals_exp = lax.dot_general(
            vals, e_exp, (((1,), (0,)), ((), ())),
            precision=lax.Precision.HIGHEST)
        cmpf = jnp.where(rows < vals_exp, jnp.float32(1.0), jnp.float32(0.0))
        cnt = lax.dot_general(
            cmpf, g_sum, (((1,), (0,)), ((), ())),
            precision=lax.Precision.HIGHEST).astype(jnp.int32)
        h = (keys * HASH_K32 + cnt) * HASH_K32
        found = keys < np.int32(n_feat)
        outk_ref[...] = jnp.where(found, h & np.int32(MASK22),
                                  keys & np.int32(MASK22))
        outv_ref[...] = jnp.where(found, jnp.float32(1.0), vals)

    return pl.pallas_call(
        body,
        grid=(grid,),
        in_specs=[
            pl.BlockSpec((TC_BR, 128), lambda i: (i, np.int32(0))),
            pl.BlockSpec((TC_BR, 8), lambda i: (i, np.int32(0))),
            pl.BlockSpec((TC_BR, 8), lambda i: (i, np.int32(0))),
        ],
        out_specs=[
            pl.BlockSpec((TC_BR, 8), lambda i: (i, np.int32(0))),
            pl.BlockSpec((TC_BR, 8), lambda i: (i, np.int32(0))),
        ],
        out_shape=[
            jax.ShapeDtypeStruct((m, 8), jnp.int32),
            jax.ShapeDtypeStruct((m, 8), jnp.float32),
        ],
    )(rows128, keys8, vals8)


def kernel(input_ids, input_keys, input_vals, feature_ids, bin_vals):
    n = input_keys.shape[0]
    n_feat = feature_ids.shape[0]
    keys32 = input_keys.astype(jnp.int32)
    table = bin_vals.reshape(n_feat, N_BIN)
    table_padded = jnp.pad(table, ((0, KEY_RANGE - n_feat), (0, 0)))
    rows = _sc_gather(keys32, table_padded)
    outk8, outv8 = _tc_bin_hash(
        rows.reshape(n // 8, 8 * N_BIN),
        keys32.reshape(n // 8, 8),
        input_vals.reshape(n // 8, 8),
        n_feat,
    )
    out_keys = outk8.reshape(n).astype(jnp.int64)
    out_vals = outv8.reshape(n)
    dense_shape = jnp.array([BATCH_DIM, 1 << OUT_BITS], dtype=jnp.int64)
    return out_keys, out_vals, input_ids, dense_shape
